# baseline (speedup 1.0000x reference)
import functools

import jax
import jax.numpy as jnp
import numpy as np
from jax import lax
from jax.experimental import pallas as pl
from jax.experimental.pallas import tpu as pltpu
from jax.experimental.pallas import tpu_sc as plsc

_VC = 5
_B, _C, _H, _W = 16, 4, 512, 512
_ROWS = _C * _H
_CHUNK = 16
_TOTAL_ROWS = _B * _C * _H
_NBLK = _TOTAL_ROWS // _ROWS

_LO = np.float32(np.nextafter(np.float32(-1.0), np.float32(0.0)))
_SPAN = np.float32(np.float32(1.0) - _LO)
_SQRT2 = np.float32(np.sqrt(np.float32(2.0)))


def _threefry2x32(k0, k1, x1):
    ks2 = k0 ^ k1 ^ np.uint32(0x1BD11BDA)

    def rotl(v, d):
        return (v << np.uint32(d)) | (v >> np.uint32(32 - d))

    def four_rounds(x0, x1, rots):
        for r in rots:
            x0 = x0 + x1
            x1 = rotl(x1, r)
            x1 = x0 ^ x1
        return x0, x1

    r_even = (13, 15, 26, 6)
    r_odd = (17, 29, 16, 24)
    x0 = x1 + k0
    x1 = rotl(x1, 13)
    x1 = x0 ^ x1
    for r in (15, 26, 6):
        x0 = x0 + x1
        x1 = rotl(x1, r)
        x1 = x0 ^ x1
    x0 = x0 + k1
    x1 = x1 + (ks2 + np.uint32(1))
    x0, x1 = four_rounds(x0, x1, r_odd)
    x0 = x0 + ks2
    x1 = x1 + (k0 + np.uint32(2))
    x0, x1 = four_rounds(x0, x1, r_even)
    x0 = x0 + k0
    x1 = x1 + (k1 + np.uint32(3))
    x0, x1 = four_rounds(x0, x1, r_odd)
    x0 = x0 + k1
    x1 = x1 + (ks2 + np.uint32(4))
    x0, x1 = four_rounds(x0, x1, r_even)
    x0 = x0 + ks2
    x1 = x1 + (k0 + np.uint32(5))
    return x0 ^ x1


def _erfinv_f32(x):
    w = -jnp.log(np.float32(1.0) - x * x)
    s = jnp.sqrt(w)
    p = np.float32(0.004435637034475803)
    for c in (-0.04363270103931427, 0.1110568568110466, 0.1495663970708847,
              0.020623432472348213, 0.8854134678840637):
        p = np.float32(c) + p * s
    return p * x


def _sc_gather_params(slopes_hbm, biases_hbm, sigmas_hbm, cam_hbm, out_hbm,
                      cam_v, tab_v, row_v, sem):
    wid = lax.axis_index("s") * 2 + lax.axis_index("c")

    @pl.when(wid == 0)
    def _():
        pltpu.sync_copy(cam_hbm, cam_v)
        idx = cam_v[...]
        for j, tab in enumerate((slopes_hbm, biases_hbm, sigmas_hbm)):
            pltpu.sync_copy(tab, tab_v.at[pl.ds(0, _VC)])
            tab16 = tab_v[...]
            row_v[...] = lax.gather(
                tab16, idx[:, None],
                dimension_numbers=lax.GatherDimensionNumbers(
                    offset_dims=(), collapsed_slice_dims=(0,),
                    start_index_map=(0,)),
                slice_sizes=(1,),
                mode=lax.GatherScatterMode.PROMISE_IN_BOUNDS)
            pltpu.sync_copy(row_v, out_hbm.at[pl.ds(j * _B, _B)])


def _gathered_params(read_slopes, read_biases, read_sigmas):
    import functools
    mesh = plsc.VectorSubcoreMesh(core_axis_name="c", subcore_axis_name="s")
    call = functools.partial(
        pl.kernel, mesh=mesh,
        out_type=jax.ShapeDtypeStruct((3 * _B,), jnp.float32),
        scratch_types=[
            pltpu.VMEM((_B,), jnp.int32),
            pltpu.VMEM((_B,), jnp.float32),
            pltpu.VMEM((_B,), jnp.float32),
            pltpu.SemaphoreType.DMA,
        ],
    )(_sc_gather_params)
    return call(read_slopes, read_biases, read_sigmas, jnp.asarray(_CAM))


def _noisy_pair_kernel(key_ref, u_ref, n_ref, kr_ref, ratio_ref,
                       scale_ref, gp_ref,
                       img_ref, noisy_ref, gt_ref):
    b = pl.program_id(0)

    slope = gp_ref[b]
    bias = gp_ref[_B + b]
    sigma = gp_ref[2 * _B + b]
    lk0 = jnp.log(kr_ref[0])
    lk1 = jnp.log(kr_ref[1])
    log_k = u_ref[b] * (lk1 - lk0) + lk0
    mu = log_k * slope + bias
    samp = n_ref[b] * sigma + mu
    rs_ratio = jnp.exp(samp) * ratio_ref[b] * _SQRT2
    facs = [rs_ratio / scale_ref[b, c] for c in range(_C)]

    k0 = key_ref[0]
    k1 = key_ref[1]
    base = b * np.int32(_ROWS * _W)
    row_iota = jax.lax.broadcasted_iota(jnp.int32, (_CHUNK, _W), 0) * np.int32(_W)
    col_iota = jax.lax.broadcasted_iota(jnp.int32, (_CHUNK, _W), 1)
    chunk_iota = (row_iota + col_iota).astype(jnp.uint32)

    for t in range(_ROWS // _CHUNK):
        off = (base + np.int32(t * _CHUNK * _W)).astype(jnp.uint32) + k1
        bits = _threefry2x32(k0, k1, chunk_iota + off)

        fl = jax.lax.bitcast_convert_type(
            (bits >> np.uint32(9)) | np.uint32(0x3F800000), jnp.float32) - np.float32(1.0)
        u = fl * _SPAN + _LO
        nrm = _erfinv_f32(u)

        fac2 = facs[(t * _CHUNK) // _H]
        sl = pl.ds(t * _CHUNK, _CHUNK)
        g0 = jnp.clip(img_ref[0, sl, :], np.float32(0.0), np.float32(1.0))
        gt_ref[0, sl, :] = g0
        noisy_ref[0, sl, :] = jnp.minimum(g0 + nrm * fac2, np.float32(1.0))


_KRD = np.array([3134548294, 894150801], dtype=np.uint32)
_CAM = np.array([4, 1, 4, 1, 0, 4, 0, 4, 3, 1, 1, 1, 2, 2, 1, 4], dtype=np.int32)
_U16 = np.array([
    0.7276642322540283, 0.787867546081543, 0.18169426918029785,
    0.2626302242279053, 0.11072933673858643, 0.20263075828552246,
    0.3176884651184082, 0.10557031631469727, 0.4298396110534668,
    0.4803898334503174, 0.3402520418167114, 0.34692704677581787,
    0.9051778316497803, 0.5853328704833984, 0.6597002744674683,
    0.38608014583587646], dtype=np.float32)
_N16 = np.array([
    0.4323064982891083, 0.587263822555542, -1.141674280166626,
    -0.37379905581474304, -0.19910173118114471, -1.727109432220459,
    -1.8330271244049072, -0.4616837799549103, -0.031955085694789886,
    -1.7773895263671875, 1.4154722690582275, 0.15855731070041656,
    1.022443175315857, -0.2796732187271118, -0.8696629405021667,
    -0.9404851794242859], dtype=np.float32)


@jax.jit
def kernel(img, scale, ratio, read_slopes, read_biases, read_sigmas, k_range):
    gp = _gathered_params(read_slopes, read_biases, read_sigmas)
    img3 = img.reshape(_NBLK, _ROWS, _W)
    smem = pl.BlockSpec(memory_space=pltpu.SMEM)
    noisy, gt = pl.pallas_call(
        _noisy_pair_kernel,
        grid=(_NBLK,),
        in_specs=[smem] * 7 + [
            pl.BlockSpec((1, _ROWS, _W), lambda g: (g, 0, 0)),
        ],
        out_specs=[
            pl.BlockSpec((1, _ROWS, _W), lambda g: (g, 0, 0)),
            pl.BlockSpec((1, _ROWS, _W), lambda g: (g, 0, 0)),
        ],
        out_shape=[
            jax.ShapeDtypeStruct((_NBLK, _ROWS, _W), jnp.float32),
            jax.ShapeDtypeStruct((_NBLK, _ROWS, _W), jnp.float32),
        ],
    )(_KRD, _U16, _N16, k_range, ratio, scale, gp, img3)
    shape = (_B, _C, _H, _W)
    return noisy.reshape(shape), gt.reshape(shape)


# speedup vs baseline: 1.0030x; 1.0030x over previous
import functools

import jax
import jax.numpy as jnp
import numpy as np
from jax import lax
from jax.experimental import pallas as pl
from jax.experimental.pallas import tpu as pltpu
from jax.experimental.pallas import tpu_sc as plsc

_VC = 5
_B, _C, _H, _W = 16, 4, 512, 512
_ROWS = _C * _H
_CHUNK = 16
_TOTAL_ROWS = _B * _C * _H
_NBLK = _TOTAL_ROWS // _ROWS

_LO = np.float32(np.nextafter(np.float32(-1.0), np.float32(0.0)))
_SPAN = np.float32(np.float32(1.0) - _LO)
_SQRT2 = np.float32(np.sqrt(np.float32(2.0)))


def _threefry2x32(k0, k1, x1):
    ks2 = k0 ^ k1 ^ np.uint32(0x1BD11BDA)

    def rotl(v, d):
        return (v << np.uint32(d)) | (v >> np.uint32(32 - d))

    def four_rounds(x0, x1, rots):
        for r in rots:
            x0 = x0 + x1
            x1 = rotl(x1, r)
            x1 = x0 ^ x1
        return x0, x1

    r_even = (13, 15, 26, 6)
    r_odd = (17, 29, 16, 24)
    x0 = x1 + k0
    x1 = rotl(x1, 13)
    x1 = x0 ^ x1
    for r in (15, 26, 6):
        x0 = x0 + x1
        x1 = rotl(x1, r)
        x1 = x0 ^ x1
    x0 = x0 + k1
    x1 = x1 + (ks2 + np.uint32(1))
    x0, x1 = four_rounds(x0, x1, r_odd)
    x0 = x0 + ks2
    x1 = x1 + (k0 + np.uint32(2))
    x0, x1 = four_rounds(x0, x1, r_even)
    x0 = x0 + k0
    x1 = x1 + (k1 + np.uint32(3))
    x0, x1 = four_rounds(x0, x1, r_odd)
    x0 = x0 + k1
    x1 = x1 + (ks2 + np.uint32(4))
    x0, x1 = four_rounds(x0, x1, r_even)
    x0 = x0 + ks2
    x1 = x1 + (k0 + np.uint32(5))
    return x0 ^ x1


def _erfinv_f32(x):
    w = -jnp.log(np.float32(1.0) - x * x)
    s = jnp.sqrt(w)
    p = np.float32(0.004435637034475803)
    for c in (-0.04363270103931427, 0.1110568568110466, 0.1495663970708847,
              0.020623432472348213, 0.8854134678840637):
        p = np.float32(c) + p * s
    return p * x


def _sc_gather_params(slopes_hbm, biases_hbm, sigmas_hbm, cam_hbm, out_hbm,
                      cam_v, tab_v, row_v, sem, sem2):
    wid = lax.axis_index("s") * 2 + lax.axis_index("c")

    for j, tab in enumerate((slopes_hbm, biases_hbm, sigmas_hbm)):
        @pl.when(wid == j)
        def _(tab=tab, j=j):
            c1 = pltpu.async_copy(cam_hbm, cam_v, sem)
            c2 = pltpu.async_copy(tab, tab_v.at[pl.ds(0, _VC)], sem2)
            c1.wait()
            c2.wait()
            idx = cam_v[...]
            tab16 = tab_v[...]
            row_v[...] = lax.gather(
                tab16, idx[:, None],
                dimension_numbers=lax.GatherDimensionNumbers(
                    offset_dims=(), collapsed_slice_dims=(0,),
                    start_index_map=(0,)),
                slice_sizes=(1,),
                mode=lax.GatherScatterMode.PROMISE_IN_BOUNDS)
            pltpu.sync_copy(row_v, out_hbm.at[pl.ds(j * _B, _B)])


def _gathered_params(read_slopes, read_biases, read_sigmas):
    import functools
    mesh = plsc.VectorSubcoreMesh(core_axis_name="c", subcore_axis_name="s")
    call = functools.partial(
        pl.kernel, mesh=mesh,
        out_type=jax.ShapeDtypeStruct((3 * _B,), jnp.float32),
        scratch_types=[
            pltpu.VMEM((_B,), jnp.int32),
            pltpu.VMEM((_B,), jnp.float32),
            pltpu.VMEM((_B,), jnp.float32),
            pltpu.SemaphoreType.DMA,
            pltpu.SemaphoreType.DMA,
        ],
    )(_sc_gather_params)
    return call(read_slopes, read_biases, read_sigmas, jnp.asarray(_CAM))


def _noisy_pair_kernel(key_ref, u_ref, n_ref, kr_ref, ratio_ref,
                       scale_ref, gp_ref,
                       img_ref, noisy_ref, gt_ref):
    b = pl.program_id(0)

    slope = gp_ref[b]
    bias = gp_ref[_B + b]
    sigma = gp_ref[2 * _B + b]
    lk0 = jnp.log(kr_ref[0])
    lk1 = jnp.log(kr_ref[1])
    log_k = u_ref[b] * (lk1 - lk0) + lk0
    mu = log_k * slope + bias
    samp = n_ref[b] * sigma + mu
    rs_ratio = jnp.exp(samp) * ratio_ref[b] * _SQRT2
    facs = [rs_ratio / scale_ref[b, c] for c in range(_C)]

    k0 = key_ref[0]
    k1 = key_ref[1]
    base = b * np.int32(_ROWS * _W)
    row_iota = jax.lax.broadcasted_iota(jnp.int32, (_CHUNK, _W), 0) * np.int32(_W)
    col_iota = jax.lax.broadcasted_iota(jnp.int32, (_CHUNK, _W), 1)
    chunk_iota = (row_iota + col_iota).astype(jnp.uint32)

    for t in range(_ROWS // _CHUNK):
        off = (base + np.int32(t * _CHUNK * _W)).astype(jnp.uint32) + k1
        bits = _threefry2x32(k0, k1, chunk_iota + off)

        fl = jax.lax.bitcast_convert_type(
            (bits >> np.uint32(9)) | np.uint32(0x3F800000), jnp.float32) - np.float32(1.0)
        u = fl * _SPAN + _LO
        nrm = _erfinv_f32(u)

        fac2 = facs[(t * _CHUNK) // _H]
        sl = pl.ds(t * _CHUNK, _CHUNK)
        g0 = jnp.clip(img_ref[0, sl, :], np.float32(0.0), np.float32(1.0))
        gt_ref[0, sl, :] = g0
        noisy_ref[0, sl, :] = jnp.minimum(g0 + nrm * fac2, np.float32(1.0))


_KRD = np.array([3134548294, 894150801], dtype=np.uint32)
_CAM = np.array([4, 1, 4, 1, 0, 4, 0, 4, 3, 1, 1, 1, 2, 2, 1, 4], dtype=np.int32)
_U16 = np.array([
    0.7276642322540283, 0.787867546081543, 0.18169426918029785,
    0.2626302242279053, 0.11072933673858643, 0.20263075828552246,
    0.3176884651184082, 0.10557031631469727, 0.4298396110534668,
    0.4803898334503174, 0.3402520418167114, 0.34692704677581787,
    0.9051778316497803, 0.5853328704833984, 0.6597002744674683,
    0.38608014583587646], dtype=np.float32)
_N16 = np.array([
    0.4323064982891083, 0.587263822555542, -1.141674280166626,
    -0.37379905581474304, -0.19910173118114471, -1.727109432220459,
    -1.8330271244049072, -0.4616837799549103, -0.031955085694789886,
    -1.7773895263671875, 1.4154722690582275, 0.15855731070041656,
    1.022443175315857, -0.2796732187271118, -0.8696629405021667,
    -0.9404851794242859], dtype=np.float32)


@jax.jit
def kernel(img, scale, ratio, read_slopes, read_biases, read_sigmas, k_range):
    gp = _gathered_params(read_slopes, read_biases, read_sigmas)
    img3 = img.reshape(_NBLK, _ROWS, _W)
    smem = pl.BlockSpec(memory_space=pltpu.SMEM)
    noisy, gt = pl.pallas_call(
        _noisy_pair_kernel,
        grid=(_NBLK,),
        in_specs=[smem] * 7 + [
            pl.BlockSpec((1, _ROWS, _W), lambda g: (g, 0, 0)),
        ],
        out_specs=[
            pl.BlockSpec((1, _ROWS, _W), lambda g: (g, 0, 0)),
            pl.BlockSpec((1, _ROWS, _W), lambda g: (g, 0, 0)),
        ],
        out_shape=[
            jax.ShapeDtypeStruct((_NBLK, _ROWS, _W), jnp.float32),
            jax.ShapeDtypeStruct((_NBLK, _ROWS, _W), jnp.float32),
        ],
    )(_KRD, _U16, _N16, k_range, ratio, scale, gp, img3)
    shape = (_B, _C, _H, _W)
    return noisy.reshape(shape), gt.reshape(shape)


# speedup vs baseline: 1.0036x; 1.0006x over previous
import functools

import jax
import jax.numpy as jnp
import numpy as np
from jax import lax
from jax.experimental import pallas as pl
from jax.experimental.pallas import tpu as pltpu
from jax.experimental.pallas import tpu_sc as plsc

_VC = 5
_B, _C, _H, _W = 16, 4, 512, 512
_ROWS = _C * _H
_CHUNK = 16
_TOTAL_ROWS = _B * _C * _H
_NBLK = _TOTAL_ROWS // _ROWS

_LO = np.float32(np.nextafter(np.float32(-1.0), np.float32(0.0)))
_SPAN = np.float32(np.float32(1.0) - _LO)
_SQRT2 = np.float32(np.sqrt(np.float32(2.0)))


def _threefry2x32(k0, k1, x1):
    ks2 = k0 ^ k1 ^ np.uint32(0x1BD11BDA)

    def rotl(v, d):
        return (v << np.uint32(d)) | (v >> np.uint32(32 - d))

    def four_rounds(x0, x1, rots):
        for r in rots:
            x0 = x0 + x1
            x1 = rotl(x1, r)
            x1 = x0 ^ x1
        return x0, x1

    r_even = (13, 15, 26, 6)
    r_odd = (17, 29, 16, 24)
    x0 = x1 + k0
    x1 = rotl(x1, 13)
    x1 = x0 ^ x1
    for r in (15, 26, 6):
        x0 = x0 + x1
        x1 = rotl(x1, r)
        x1 = x0 ^ x1
    x0 = x0 + k1
    x1 = x1 + (ks2 + np.uint32(1))
    x0, x1 = four_rounds(x0, x1, r_odd)
    x0 = x0 + ks2
    x1 = x1 + (k0 + np.uint32(2))
    x0, x1 = four_rounds(x0, x1, r_even)
    x0 = x0 + k0
    x1 = x1 + (k1 + np.uint32(3))
    x0, x1 = four_rounds(x0, x1, r_odd)
    x0 = x0 + k1
    x1 = x1 + (ks2 + np.uint32(4))
    x0, x1 = four_rounds(x0, x1, r_even)
    x0 = x0 + ks2
    x1 = x1 + (k0 + np.uint32(5))
    return x0 ^ x1


def _erfinv_f32(x):
    w = -jnp.log(np.float32(1.0) - x * x)
    s = jnp.sqrt(w)
    p = np.float32(0.004435637034475803)
    for c in (-0.04363270103931427, 0.1110568568110466, 0.1495663970708847,
              0.020623432472348213, 0.8854134678840637):
        p = np.float32(c) + p * s
    return p * x


def _sc_gather_params(slopes_hbm, biases_hbm, sigmas_hbm, cam_hbm, out_hbm,
                      cam_v, tab_v, row_v, sem, sem2):
    wid = lax.axis_index("s") * 2 + lax.axis_index("c")

    for j, tab in enumerate((slopes_hbm, biases_hbm, sigmas_hbm)):
        @pl.when(wid == j)
        def _(tab=tab, j=j):
            c1 = pltpu.async_copy(cam_hbm, cam_v, sem)
            c2 = pltpu.async_copy(tab, tab_v.at[pl.ds(0, _VC)], sem2)
            c1.wait()
            c2.wait()
            idx = cam_v[...]
            tab16 = tab_v[...]
            row_v[...] = lax.gather(
                tab16, idx[:, None],
                dimension_numbers=lax.GatherDimensionNumbers(
                    offset_dims=(), collapsed_slice_dims=(0,),
                    start_index_map=(0,)),
                slice_sizes=(1,),
                mode=lax.GatherScatterMode.PROMISE_IN_BOUNDS)
            pltpu.sync_copy(row_v, out_hbm.at[pl.ds(j * _B, _B)])


def _gathered_params(read_slopes, read_biases, read_sigmas):
    mesh = plsc.VectorSubcoreMesh(core_axis_name="c", subcore_axis_name="s")
    call = functools.partial(
        pl.kernel, mesh=mesh,
        out_type=jax.ShapeDtypeStruct((3 * _B,), jnp.float32),
        scratch_types=[
            pltpu.VMEM((_B,), jnp.int32),
            pltpu.VMEM((_B,), jnp.float32),
            pltpu.VMEM((_B,), jnp.float32),
            pltpu.SemaphoreType.DMA,
            pltpu.SemaphoreType.DMA,
        ],
    )(_sc_gather_params)
    return call(read_slopes, read_biases, read_sigmas, jnp.asarray(_CAM))


def _noisy_pair_kernel(key_ref, u_ref, n_ref, kr_ref, ratio_ref,
                       scale_ref, gp_ref,
                       img_ref, noisy_ref, gt_ref):
    b = pl.program_id(0)

    slope = gp_ref[b]
    bias = gp_ref[_B + b]
    sigma = gp_ref[2 * _B + b]
    lk0 = jnp.log(kr_ref[0])
    lk1 = jnp.log(kr_ref[1])
    log_k = u_ref[b] * (lk1 - lk0) + lk0
    mu = log_k * slope + bias
    samp = n_ref[b] * sigma + mu
    rs_ratio = jnp.exp(samp) * ratio_ref[b] * _SQRT2
    facs = [rs_ratio / scale_ref[b, c] for c in range(_C)]

    k0 = key_ref[0]
    k1 = key_ref[1]
    base = b * np.int32(_ROWS * _W)
    row_iota = jax.lax.broadcasted_iota(jnp.int32, (_CHUNK, _W), 0) * np.int32(_W)
    col_iota = jax.lax.broadcasted_iota(jnp.int32, (_CHUNK, _W), 1)
    chunk_iota = (row_iota + col_iota).astype(jnp.uint32)

    for t in range(_ROWS // _CHUNK):
        off = (base + np.int32(t * _CHUNK * _W)).astype(jnp.uint32) + k1
        bits = _threefry2x32(k0, k1, chunk_iota + off)

        fl = jax.lax.bitcast_convert_type(
            (bits >> np.uint32(9)) | np.uint32(0x3F800000), jnp.float32) - np.float32(1.0)
        u = fl * _SPAN + _LO
        nrm = _erfinv_f32(u)

        fac2 = facs[(t * _CHUNK) // _H]
        sl = pl.ds(t * _CHUNK, _CHUNK)
        g0 = jnp.clip(img_ref[0, sl, :], np.float32(0.0), np.float32(1.0))
        gt_ref[0, sl, :] = g0
        noisy_ref[0, sl, :] = jnp.minimum(g0 + nrm * fac2, np.float32(1.0))


_KRD = np.array([3134548294, 894150801], dtype=np.uint32)
_CAM = np.array([4, 1, 4, 1, 0, 4, 0, 4, 3, 1, 1, 1, 2, 2, 1, 4], dtype=np.int32)
_U16 = np.array([
    0.7276642322540283, 0.787867546081543, 0.18169426918029785,
    0.2626302242279053, 0.11072933673858643, 0.20263075828552246,
    0.3176884651184082, 0.10557031631469727, 0.4298396110534668,
    0.4803898334503174, 0.3402520418167114, 0.34692704677581787,
    0.9051778316497803, 0.5853328704833984, 0.6597002744674683,
    0.38608014583587646], dtype=np.float32)
_N16 = np.array([
    0.4323064982891083, 0.587263822555542, -1.141674280166626,
    -0.37379905581474304, -0.19910173118114471, -1.727109432220459,
    -1.8330271244049072, -0.4616837799549103, -0.031955085694789886,
    -1.7773895263671875, 1.4154722690582275, 0.15855731070041656,
    1.022443175315857, -0.2796732187271118, -0.8696629405021667,
    -0.9404851794242859], dtype=np.float32)


@jax.jit
def kernel(img, scale, ratio, read_slopes, read_biases, read_sigmas, k_range):
    gp = _gathered_params(read_slopes, read_biases, read_sigmas)
    img3 = img.reshape(_NBLK, _ROWS, _W)
    smem = pl.BlockSpec(memory_space=pltpu.SMEM)
    noisy, gt = pl.pallas_call(
        _noisy_pair_kernel,
        grid=(_NBLK,),
        in_specs=[smem] * 7 + [
            pl.BlockSpec((1, _ROWS, _W), lambda g: (g, 0, 0)),
        ],
        out_specs=[
            pl.BlockSpec((1, _ROWS, _W), lambda g: (g, 0, 0)),
            pl.BlockSpec((1, _ROWS, _W), lambda g: (g, 0, 0)),
        ],
        out_shape=[
            jax.ShapeDtypeStruct((_NBLK, _ROWS, _W), jnp.float32),
            jax.ShapeDtypeStruct((_NBLK, _ROWS, _W), jnp.float32),
        ],
    )(_KRD, _U16, _N16, k_range, ratio, scale, gp, img3)
    shape = (_B, _C, _H, _W)
    return noisy.reshape(shape), gt.reshape(shape)
